# Initial kernel scaffold; baseline (speedup 1.0000x reference)
import jax
import jax.numpy as jnp
from jax.experimental import pallas as pl


def kernel(x, W_enc, b_enc, embeddings, W_dq, b_dq, W_de, b_de):
    raise NotImplementedError("write your pallas kernel here")


# speedup vs baseline: 1.0953x; 1.0953x over previous
import functools

import jax
import jax.numpy as jnp
from jax import lax
from jax.experimental import pallas as pl
from jax.experimental.pallas import tpu as pltpu
from jax.experimental.pallas import tpu_sc as plsc

SOM0, SOM1 = 64, 128
K = SOM0 * SOM1
LATENT = 256
IN_DIM = 1024
B = 4096

BB = 128
NB = B // BB
DEC_BB = K // NB
KPAD = 8
ZROW = K

NC, NS = 2, 16
NW = NC * NS
BPW = B // NW
XCH = 64


def _tc_body(x_ref, we_ref, be_ref, emb_ref, wde_ref, bde_ref, wdq_ref,
             bdq_ref, ze_ref, dist_ref, k_ref, xe_ref, dec_ref):
    i = pl.program_id(0)
    z = jnp.dot(x_ref[...], we_ref[...],
                preferred_element_type=jnp.float32) + be_ref[...]
    ze_ref[...] = z
    emb = emb_ref[...]
    e2 = jnp.sum(emb * emb, axis=1)
    z2 = jnp.sum(z * z, axis=1, keepdims=True)
    cross = lax.dot_general(z, emb, (((1,), (1,)), ((), ())),
                            preferred_element_type=jnp.float32)
    dist = z2 - 2.0 * cross + e2[None, :]
    dist_ref[...] = dist
    m = jnp.min(dist, axis=1, keepdims=True)
    ids = lax.broadcasted_iota(jnp.int32, dist.shape, 1)
    k_ref[0, 0, :] = jnp.min(jnp.where(dist == m, ids, K), axis=1)
    xe_ref[...] = jnp.dot(z, wde_ref[...],
                          preferred_element_type=jnp.float32) + bde_ref[...]
    eslice = emb_ref[pl.ds(i * DEC_BB, DEC_BB), :]
    dec_ref[...] = jnp.dot(eslice, wdq_ref[...],
                           preferred_element_type=jnp.float32) + bdq_ref[...]


_tc_call = pl.pallas_call(
    _tc_body,
    grid=(NB,),
    in_specs=[
        pl.BlockSpec((BB, IN_DIM), lambda i: (i, 0)),
        pl.BlockSpec((IN_DIM, LATENT), lambda i: (0, 0)),
        pl.BlockSpec((1, LATENT), lambda i: (0, 0)),
        pl.BlockSpec((K, LATENT), lambda i: (0, 0)),
        pl.BlockSpec((LATENT, IN_DIM), lambda i: (0, 0)),
        pl.BlockSpec((1, IN_DIM), lambda i: (0, 0)),
        pl.BlockSpec((LATENT, IN_DIM), lambda i: (0, 0)),
        pl.BlockSpec((1, IN_DIM), lambda i: (0, 0)),
    ],
    out_specs=[
        pl.BlockSpec((BB, LATENT), lambda i: (i, 0)),
        pl.BlockSpec((BB, K), lambda i: (i, 0)),
        pl.BlockSpec((1, 1, BB), lambda i: (i, 0, 0)),
        pl.BlockSpec((BB, IN_DIM), lambda i: (i, 0)),
        pl.BlockSpec((DEC_BB, IN_DIM), lambda i: (i, 0)),
    ],
    out_shape=[
        jax.ShapeDtypeStruct((B, LATENT), jnp.float32),
        jax.ShapeDtypeStruct((B, K), jnp.float32),
        jax.ShapeDtypeStruct((NB, 1, BB), jnp.int32),
        jax.ShapeDtypeStruct((B, IN_DIM), jnp.float32),
        jax.ShapeDtypeStruct((K, IN_DIM), jnp.float32),
    ],
)


@functools.cache
def _make_sc_gather():
  @functools.partial(
    pl.kernel,
    mesh=plsc.VectorSubcoreMesh(core_axis_name="c", subcore_axis_name="s"),
    out_type=[
        jax.ShapeDtypeStruct((B, LATENT), jnp.float32),
        jax.ShapeDtypeStruct((B, LATENT), jnp.float32),
        jax.ShapeDtypeStruct((B, LATENT), jnp.float32),
        jax.ShapeDtypeStruct((B, LATENT), jnp.float32),
        jax.ShapeDtypeStruct((B, IN_DIM), jnp.float32),
    ],
    scratch_types=[
        pltpu.VMEM((BPW,), jnp.int32),
        pltpu.VMEM((BPW,), jnp.int32),
        pltpu.VMEM((BPW,), jnp.int32),
        pltpu.VMEM((BPW,), jnp.int32),
        pltpu.VMEM((XCH,), jnp.int32),
        pltpu.VMEM((BPW, LATENT), jnp.float32),
        pltpu.VMEM((XCH, IN_DIM), jnp.float32),
        pltpu.SemaphoreType.DMA,
    ],
  )
  def _sc_gather(table_hbm, dec_hbm, k_hbm, zq_hbm, zup_hbm, zdn_hbm, zlf_hbm,
                 xq_hbm, kv, iu, idn, ilf, kx, rows, xrows, sem):
    wid = lax.axis_index("s") * NC + lax.axis_index("c")
    base = wid * BPW
    pltpu.sync_copy(k_hbm.at[pl.ds(base, BPW)], kv)
    for c in range(BPW // 16):
        sl = pl.ds(c * 16, 16)
        kk = kv[sl]
        k1 = lax.shift_right_logical(kk, 7)
        k2 = jnp.bitwise_and(kk, SOM1 - 1)
        iu[sl] = jnp.where(k1 < SOM0 - 1, kk + SOM1, ZROW)
        idn[sl] = jnp.where(k1 > 0, kk - SOM1, ZROW)
        ilf[sl] = jnp.where(k2 > 0, kk - 1, ZROW)
    pltpu.async_copy(table_hbm.at[kv], rows, sem).wait()
    pltpu.sync_copy(rows, zq_hbm.at[pl.ds(base, BPW)])
    pltpu.async_copy(table_hbm.at[iu], rows, sem).wait()
    pltpu.sync_copy(rows, zup_hbm.at[pl.ds(base, BPW)])
    pltpu.async_copy(table_hbm.at[idn], rows, sem).wait()
    pltpu.sync_copy(rows, zdn_hbm.at[pl.ds(base, BPW)])
    pltpu.async_copy(table_hbm.at[ilf], rows, sem).wait()
    pltpu.sync_copy(rows, zlf_hbm.at[pl.ds(base, BPW)])
    for c in range(BPW // XCH):
        for j in range(XCH // 16):
            kx[pl.ds(j * 16, 16)] = kv[pl.ds(c * XCH + j * 16, 16)]
        pltpu.async_copy(dec_hbm.at[kx], xrows, sem).wait()
        pltpu.sync_copy(xrows, xq_hbm.at[pl.ds(base + c * XCH, XCH)])

  return _sc_gather


def kernel(x, W_enc, b_enc, embeddings, W_dq, b_dq, W_de, b_de):
    emb_flat = embeddings.reshape(K, LATENT)
    z_e, z_dist, k_blk, x_e, dec = _tc_call(
        x, W_enc, b_enc.reshape(1, LATENT), emb_flat,
        W_de, b_de.reshape(1, IN_DIM), W_dq, b_dq.reshape(1, IN_DIM))
    k = k_blk.reshape(B)
    table = jnp.concatenate(
        [emb_flat, jnp.zeros((KPAD, LATENT), jnp.float32)], axis=0)
    z_q, z_up, z_dn, z_lf, x_q = _make_sc_gather()(table, dec, k)
    z_rt = jnp.zeros((B, LATENT), jnp.float32)
    z_q_neighbors = jnp.stack([z_q, z_up, z_dn, z_rt, z_lf], axis=1)
    return (x_e, x_q, z_e, z_q, z_q_neighbors, k, z_dist)
